# baseline (speedup 1.0000x reference)
import functools

import jax
import jax.numpy as jnp
from jax import lax
from jax.experimental import pallas as pl
from jax.experimental.pallas import tpu as pltpu
from jax.experimental.pallas import tpu_sc as plsc

D = 64
K = 16
N = 50000
NW = 32
N_PAD = 51200
PW = N_PAD // NW
CH = 8
CPW = PW // CH
IDXR = PW * K // 128
TROWS = N_PAD // 4
TPT = TROWS // 16
NB = 1024
GRID = N_PAD // NB
_INV_SQRT2 = 0.7071067811865476


def _mm_body(xt_ref, wy_ref, wz_ref, y_ref, z_ref):
    xb = xt_ref[...]
    y_ref[...] = jnp.dot(xb, wy_ref[...], preferred_element_type=jnp.float32)
    z_ref[...] = jnp.dot(xb, wz_ref[...], preferred_element_type=jnp.float32)


def _decode_keys(k):
    b = jnp.where(k >= 0x8000, k - 0x8000, 0xFFFF - k)
    return lax.bitcast_convert_type(b << 16, jnp.float32)


def _stats_body(m_ref, z_ref, s_ref):
    i = pl.program_id(0)
    t = _decode_keys(m_ref[...][:, :D]) + z_ref[...]
    rows = lax.broadcasted_iota(jnp.int32, t.shape, 0) + i * NB
    t = jnp.where(rows < N, t, 0.0)
    part = jnp.concatenate(
        [jnp.sum(t, axis=0, keepdims=True),
         jnp.sum(t * t, axis=0, keepdims=True)], axis=0)

    @pl.when(i == 0)
    def _():
        s_ref[...] = jnp.zeros_like(s_ref)

    s_ref[...] += part


def _bn_body(m_ref, z_ref, s_ref, g_ref, b_ref, o_ref):
    inv_n = 1.0 / N
    mean = s_ref[0:1, :] * inv_n
    var = s_ref[1:2, :] * inv_n - mean * mean
    scale = g_ref[...] * lax.rsqrt(var + 1e-5)
    shift = b_ref[...] - mean * scale
    t = (_decode_keys(m_ref[...][:, :D]) + z_ref[...]) * scale + shift
    o_ref[...] = t * 0.5 * (1.0 + lax.erf(t * _INV_SQRT2))


def _gather_max_body(tab_hbm, idx_hbm, par_hbm, out_hbm,
                     tab_sh, idxb, parb, buf, m_v, sem):
    sid = lax.axis_index("s")
    core = lax.axis_index("c")
    wid = sid * 2 + core
    base = wid * PW
    pltpu.sync_copy(tab_hbm.at[pl.ds(pl.multiple_of(sid * TPT, 8), TPT)],
                    tab_sh.at[pl.ds(pl.multiple_of(sid * TPT, 8), TPT)])
    plsc.subcore_barrier()

    def s_body(s8, car):
        off = pl.multiple_of(wid * IDXR + s8 * 8, 8)
        pltpu.sync_copy(idx_hbm.at[pl.ds(off, 8)], idxb)
        pltpu.sync_copy(par_hbm.at[pl.ds(off, 8)], parb)

        def c_body(cc, car2):
            pltpu.async_copy(tab_sh.at[idxb.at[cc]], buf, sem)
            pltpu.make_async_copy(tab_sh.at[pl.ds(0, CH * K)], buf,
                                  sem).wait()
            lo_mask = jnp.int32(0xFFFF)
            for p in range(CH):
                r0 = p * K
                pv = parb[cc, pl.ds(r0, 16)]
                accs = [None] * 4
                for j in range(K):
                    q = pv[j]
                    for h in range(2):
                        v = buf[r0 + j, pl.ds(q + h * 16, 16)]
                        lo = v & lo_mask
                        hi = lax.shift_right_logical(v, 16)
                        for i, e in ((2 * h, lo), (2 * h + 1, hi)):
                            accs[i] = e if j == 0 else jnp.maximum(accs[i], e)
                m_v[p, pl.ds(0, 16)] = accs[0]
                m_v[p, pl.ds(16, 16)] = accs[1]
                m_v[p, pl.ds(32, 16)] = accs[2]
                m_v[p, pl.ds(48, 16)] = accs[3]
            c = s8 * 8 + cc
            pltpu.sync_copy(
                m_v, out_hbm.at[pl.ds(pl.multiple_of(base + c * CH, 8), CH)])
            return car2

        lax.fori_loop(0, 8, c_body, 0)
        return car

    lax.fori_loop(0, IDXR // 8, s_body, 0)


@functools.cache
def _gather_max():
    mesh = plsc.VectorSubcoreMesh(core_axis_name="c", subcore_axis_name="s")
    return pl.kernel(
        _gather_max_body,
        mesh=mesh,
        out_type=jax.ShapeDtypeStruct((N_PAD, 128), jnp.int32),
        scratch_types=[
            pltpu.VMEM_SHARED((TROWS, 128), jnp.int32),
            pltpu.VMEM((8, 128), jnp.int32),
            pltpu.VMEM((8, 128), jnp.int32),
            pltpu.VMEM((CH * K, 128), jnp.int32),
            pltpu.VMEM((CH, 128), jnp.int32),
            pltpu.SemaphoreType.DMA,
        ],
    )


def kernel(x, neighbor_ind, W, gamma, beta):
    n = x.shape[2]
    xt = jnp.pad(x[0].T, ((0, N_PAD - n), (0, 0)))
    nbr = jnp.pad(neighbor_ind[0].astype(jnp.int32),
                  ((0, N_PAD - n), (0, 0)))
    idx4 = (nbr // 4).reshape(N_PAD // 8, 128)
    par32 = ((nbr % 4) * 32).reshape(N_PAD // 8, 128)
    g = ([2 * i for i in range(16)] + [2 * i + 1 for i in range(16)]
         + [32 + 2 * i for i in range(16)] + [33 + 2 * i for i in range(16)])
    inv_g = [0] * D
    for l, c in enumerate(g):
        inv_g[c] = l
    gj = jnp.asarray(g)
    wy = W[:, :D].T
    wz = (W[:, D:] - W[:, :D]).T[:, gj]
    gamma = gamma[gj]
    beta = beta[gj]

    y, z = pl.pallas_call(
        _mm_body,
        grid=(GRID,),
        in_specs=[
            pl.BlockSpec((NB, D), lambda i: (i, 0)),
            pl.BlockSpec((D, D), lambda i: (0, 0)),
            pl.BlockSpec((D, D), lambda i: (0, 0)),
        ],
        out_specs=[pl.BlockSpec((NB, D), lambda i: (i, 0)),
                   pl.BlockSpec((NB, D), lambda i: (i, 0))],
        out_shape=[jax.ShapeDtypeStruct((N_PAD, D), jnp.float32)] * 2,
    )(xt, wy, wz)

    yb = lax.bitcast_convert_type(y.astype(jnp.bfloat16),
                                  jnp.uint16).astype(jnp.int32)
    keys = jnp.where(yb < 0x8000, yb + 0x8000, 0xFFFF - yb)
    kp = keys.reshape(N_PAD // 4, 128, 2)
    tab = kp[..., 0] | (kp[..., 1] << 16)

    m = _gather_max()(tab, idx4, par32)

    s = pl.pallas_call(
        _stats_body,
        grid=(GRID,),
        in_specs=[pl.BlockSpec((NB, 128), lambda i: (i, 0)),
                  pl.BlockSpec((NB, D), lambda i: (i, 0))],
        out_specs=pl.BlockSpec((2, D), lambda i: (0, 0)),
        out_shape=jax.ShapeDtypeStruct((2, D), jnp.float32),
    )(m, z)

    out = pl.pallas_call(
        _bn_body,
        grid=(GRID,),
        in_specs=[pl.BlockSpec((NB, 128), lambda i: (i, 0)),
                  pl.BlockSpec((NB, D), lambda i: (i, 0)),
                  pl.BlockSpec((2, D), lambda i: (0, 0)),
                  pl.BlockSpec((1, D), lambda i: (0, 0)),
                  pl.BlockSpec((1, D), lambda i: (0, 0))],
        out_specs=pl.BlockSpec((NB, D), lambda i: (i, 0)),
        out_shape=jax.ShapeDtypeStruct((N_PAD, D), jnp.float32),
    )(m, z, s, gamma.reshape(1, D), beta.reshape(1, D))

    return out[:n, jnp.asarray(inv_g)].T[None]
